# baseline (speedup 1.0000x reference)
import jax
import jax.numpy as jnp
from jax.experimental import pallas as pl

N = 100000
E = 3200000
D_IN = 128
D_E = 16
D_OUT = 128

_BLK = 1024
_PB = _BLK * D_E // 128
_PROWS = N * D_E // 128


def _mlp_body(nf_ref, sp_ref, w1a_ref, w1e_ref, w2a_ref, w2e_ref,
              b1_ref, b2_ref, o_ref):
    nf = nf_ref[...]
    pm = sp_ref[...]
    hb1 = (pm @ w1e_ref[...]).reshape(_BLK, D_OUT)
    hb2 = (pm @ w2e_ref[...]).reshape(_BLK, D_OUT)
    h1 = jnp.maximum(nf @ w1a_ref[...] + hb1 + b1_ref[...], 0.0)
    h2 = jnp.maximum(h1 @ w2a_ref[...] + hb2 + b2_ref[...], 0.0)
    o_ref[...] = h2


def _tc_mlp(nf2, segp, w1a, w1e, w2a, w2e, b1r, b2r):
    grid = (N + _BLK - 1) // _BLK
    full = lambda i: (0, 0)
    return pl.pallas_call(
        _mlp_body,
        grid=(grid,),
        in_specs=[
            pl.BlockSpec((_BLK, D_IN), lambda i: (i, 0)),
            pl.BlockSpec((_PB, 128), lambda i: (i, 0)),
            pl.BlockSpec((D_IN, D_OUT), full),
            pl.BlockSpec((128, 8 * D_OUT), full),
            pl.BlockSpec((D_OUT, D_OUT), full),
            pl.BlockSpec((128, 8 * D_OUT), full),
            pl.BlockSpec((1, D_OUT), full),
            pl.BlockSpec((1, D_OUT), full),
        ],
        out_specs=pl.BlockSpec((_BLK, D_OUT), lambda i: (i, 0)),
        out_shape=jax.ShapeDtypeStruct((N, D_OUT), jnp.float32),
    )(nf2, segp, w1a, w1e, w2a, w2e, b1r, b2r)


def kernel(nfeats, edge_index, efeats, W1, b1, W2, b2):
    dst = edge_index[1]
    ef = efeats.reshape(E, D_E)
    seg = jax.ops.segment_sum(ef, dst, num_segments=N)
    cnt = jax.ops.segment_sum(jnp.ones((E,), jnp.float32), dst, num_segments=N)
    segp = (seg / jnp.maximum(cnt, 1.0)[:, None]).reshape(_PROWS, 128)

    nf2 = nfeats.reshape(N, D_IN)
    w1a = W1[:, :D_IN].T
    w1e = jnp.kron(jnp.eye(8, dtype=jnp.float32), W1[:, D_IN:].T)
    w2a = W2[:, :D_OUT].T
    w2e = jnp.kron(jnp.eye(8, dtype=jnp.float32), W2[:, D_OUT:].T)
    out = _tc_mlp(nf2, segp, w1a, w1e, w2a, w2e,
                  b1.reshape(1, D_OUT), b2.reshape(1, D_OUT))
    return out


# speedup vs baseline: 1.0033x; 1.0033x over previous
import jax
import jax.numpy as jnp
from jax.experimental import pallas as pl

N = 100000
E = 3200000
D_IN = 128
D_E = 16
D_OUT = 128

_BLK = 1024
_PB = _BLK * D_E // 128
_PROWS = N * D_E // 128


def _mlp_body(nf_ref, sp_ref, cp_ref, w1a_ref, w1e_ref, w2a_ref, w2e_ref,
              b1_ref, b2_ref, o_ref):
    nf = nf_ref[...]
    pm = sp_ref[...] / jnp.maximum(cp_ref[...], 1.0)
    hb1 = (pm @ w1e_ref[...]).reshape(_BLK, D_OUT)
    hb2 = (pm @ w2e_ref[...]).reshape(_BLK, D_OUT)
    h1 = jnp.maximum(nf @ w1a_ref[...] + hb1 + b1_ref[...], 0.0)
    h2 = jnp.maximum(h1 @ w2a_ref[...] + hb2 + b2_ref[...], 0.0)
    o_ref[...] = h2


def _tc_mlp(nf2, segp, cntp, w1a, w1e, w2a, w2e, b1r, b2r):
    grid = (N + _BLK - 1) // _BLK
    full = lambda i: (0, 0)
    return pl.pallas_call(
        _mlp_body,
        grid=(grid,),
        in_specs=[
            pl.BlockSpec((_BLK, D_IN), lambda i: (i, 0)),
            pl.BlockSpec((_PB, 128), lambda i: (i, 0)),
            pl.BlockSpec((_PB, 128), lambda i: (i, 0)),
            pl.BlockSpec((D_IN, D_OUT), full),
            pl.BlockSpec((128, 8 * D_OUT), full),
            pl.BlockSpec((D_OUT, D_OUT), full),
            pl.BlockSpec((128, 8 * D_OUT), full),
            pl.BlockSpec((1, D_OUT), full),
            pl.BlockSpec((1, D_OUT), full),
        ],
        out_specs=pl.BlockSpec((_BLK, D_OUT), lambda i: (i, 0)),
        out_shape=jax.ShapeDtypeStruct((N, D_OUT), jnp.float32),
    )(nf2, segp, cntp, w1a, w1e, w2a, w2e, b1r, b2r)


def kernel(nfeats, edge_index, efeats, W1, b1, W2, b2):
    dst = edge_index[1]
    ef = efeats.reshape(E, D_E)
    seg = jax.ops.segment_sum(ef, dst, num_segments=N)
    cnt = jax.ops.segment_sum(jnp.ones((E,), jnp.float32), dst, num_segments=N)
    segp = seg.reshape(_PROWS, 128)
    cntp = jnp.repeat(cnt, D_E).reshape(_PROWS, 128)

    nf2 = nfeats.reshape(N, D_IN)
    w1a = W1[:, :D_IN].T
    w1e = jnp.kron(jnp.eye(8, dtype=jnp.float32), W1[:, D_IN:].T)
    w2a = W2[:, :D_OUT].T
    w2e = jnp.kron(jnp.eye(8, dtype=jnp.float32), W2[:, D_OUT:].T)
    out = _tc_mlp(nf2, segp, cntp, w1a, w1e, w2a, w2e,
                  b1.reshape(1, D_OUT), b2.reshape(1, D_OUT))
    return out
